# Initial kernel scaffold; baseline (speedup 1.0000x reference)
import jax
import jax.numpy as jnp
from jax.experimental import pallas as pl


def kernel(label_pre, bbox_pre, proposals):
    raise NotImplementedError("write your pallas kernel here")


# speedup vs baseline: 3.3293x; 3.3293x over previous
import functools

import jax
import jax.numpy as jnp
import numpy as np
from jax import lax
from jax.experimental import pallas as pl
from jax.experimental.pallas import tpu as pltpu

N = 20000
C = 80
SCORE_THRESH = 0.01
NMS_THRESH = 0.5
DET_PER_IM = 100
PRE_NMS_TOPK = 1000
BBOX_CLIP = float(np.log(1000.0 / 16.0))

_ROWS = 1000
_PAD = 1024


def _score_kernel(lab_ref, dx_ref, dy_ref, dw_ref, dh_ref, prop_ref, out_ref):
    lab = lab_ref[...]
    m = jnp.max(lab, axis=1, keepdims=True)
    e = jnp.exp(lab - m)
    s = jnp.sum(e, axis=1, keepdims=True)
    scores = (e / s)[:, 1:]

    x1 = prop_ref[:, 0:1]
    y1 = prop_ref[:, 1:2]
    x2 = prop_ref[:, 2:3]
    y2 = prop_ref[:, 3:4]
    w = x2 - x1
    h = y2 - y1
    cx = x1 + 0.5 * w
    cy = y1 + 0.5 * h
    dx = dx_ref[...]
    dy = dy_ref[...]
    dwc = jnp.minimum(dw_ref[...], BBOX_CLIP)
    dhc = jnp.minimum(dh_ref[...], BBOX_CLIP)
    pcx = dx * w + cx
    pcy = dy * h + cy
    pw = jnp.exp(dwc) * w
    ph = jnp.exp(dhc) * h
    ax = (pcx + 0.5 * pw) - (pcx - 0.5 * pw)
    ay = (pcy + 0.5 * ph) - (pcy - 0.5 * ph)
    area = ay * ax
    valid = (scores > SCORE_THRESH) & (area > 0.1)
    out_ref[...] = jnp.where(valid, scores, -1.0)


def _nms_kernel(bx1_ref, by1_ref, bx2_ref, by2_ref, lab_ref, sc_ref, out_ref):
    jj = (lax.broadcasted_iota(jnp.int32, (8, 128), 0) * 128
          + lax.broadcasted_iota(jnp.int32, (8, 128), 1))
    real = jj < PRE_NMS_TOPK
    bx1 = bx1_ref[...]
    by1 = by1_ref[...]
    bx2 = bx2_ref[...]
    by2 = by2_ref[...]
    labs = lab_ref[...]
    scores = sc_ref[...]

    neg = jnp.float32(-jnp.inf)
    mc = jnp.maximum(
        jnp.max(jnp.where(real, bx1, neg)),
        jnp.maximum(
            jnp.max(jnp.where(real, by1, neg)),
            jnp.maximum(jnp.max(jnp.where(real, bx2, neg)),
                        jnp.max(jnp.where(real, by2, neg)))),
    ) + 1.0
    offs = labs * mc
    ox1 = bx1 + offs
    oy1 = by1 + offs
    ox2 = bx2 + offs
    oy2 = by2 + offs
    areas = (ox2 - ox1) * (oy2 - oy1)

    def ext(sel, a):
        return jnp.sum(jnp.where(sel, a, 0.0))

    def body(i, work):
        m = jnp.max(work)
        idx = jnp.min(jnp.where(work == m, jj, jnp.int32(1 << 30)))
        sel = jj == idx
        vx1 = ext(sel, ox1)
        vy1 = ext(sel, oy1)
        vx2 = ext(sel, ox2)
        vy2 = ext(sel, oy2)
        va = ext(sel, areas)
        w_i = jnp.sum(jnp.where(sel, work, 0.0))
        valid = w_i > 0.0
        xx1 = jnp.maximum(ox1, vx1)
        yy1 = jnp.maximum(oy1, vy1)
        xx2 = jnp.minimum(ox2, vx2)
        yy2 = jnp.minimum(oy2, vy2)
        inter = jnp.clip(xx2 - xx1, 0.0) * jnp.clip(yy2 - yy1, 0.0)
        iou = inter / (areas + va - inter + 1e-9)
        work = jnp.where(iou > NMS_THRESH, neg, work)
        li = lax.broadcasted_iota(jnp.int32, (1, 128), 1)
        row = jnp.zeros((1, 128), jnp.float32)
        for k, v in enumerate(
                (ext(sel, bx1), ext(sel, by1), ext(sel, bx2), ext(sel, by2),
                 jnp.sum(jnp.where(sel, scores, 0.0)))):
            row = jnp.where(li == k, jnp.where(valid, v, 0.0), row)
        out_ref[pl.ds(i, 1), :] = row
        return work

    lax.fori_loop(0, DET_PER_IM, body, scores)


def _masked_scores(label_pre, bbox_pre, proposals, interpret):
    dx = bbox_pre[:, 4::4]
    dy = bbox_pre[:, 5::4]
    dw = bbox_pre[:, 6::4]
    dh = bbox_pre[:, 7::4]
    grid = (N // _ROWS,)
    rb = lambda j: (j, 0)
    return pl.pallas_call(
        _score_kernel,
        grid=grid,
        in_specs=[
            pl.BlockSpec((_ROWS, C + 1), rb),
            pl.BlockSpec((_ROWS, C), rb),
            pl.BlockSpec((_ROWS, C), rb),
            pl.BlockSpec((_ROWS, C), rb),
            pl.BlockSpec((_ROWS, C), rb),
            pl.BlockSpec((_ROWS, 4), rb),
        ],
        out_specs=pl.BlockSpec((_ROWS, C), rb),
        out_shape=jax.ShapeDtypeStruct((N, C), jnp.float32),
        interpret=interpret,
    )(label_pre, dx, dy, dw, dh, proposals)


def _nms(top_boxes, top_labels_f, top_scores, interpret):
    npad = _PAD - PRE_NMS_TOPK

    def padv(a, v):
        return jnp.concatenate([a, jnp.full((npad,), v, a.dtype)]).reshape(8, 128)

    args = (
        padv(top_boxes[:, 0], 0.0),
        padv(top_boxes[:, 1], 0.0),
        padv(top_boxes[:, 2], 0.0),
        padv(top_boxes[:, 3], 0.0),
        padv(top_labels_f, 0.0),
        padv(top_scores, -jnp.inf),
    )
    out = pl.pallas_call(
        _nms_kernel,
        in_specs=[pl.BlockSpec((8, 128), lambda: (0, 0))] * 6,
        out_specs=pl.BlockSpec((DET_PER_IM, 128), lambda: (0, 0)),
        out_shape=jax.ShapeDtypeStruct((DET_PER_IM, 128), jnp.float32),
        interpret=interpret,
    )(*args)
    return out[:, :5]


def _impl(label_pre, bbox_pre, proposals, interpret=False):
    scores_m = _masked_scores(label_pre, bbox_pre, proposals, interpret)
    flat = scores_m.reshape(-1)
    top_scores, top_idx = lax.top_k(flat, PRE_NMS_TOPK)
    n = top_idx // C
    c = top_idx % C
    deltas = bbox_pre[n[:, None], (4 * (c + 1))[:, None] + jnp.arange(4)[None, :]]
    boxes = proposals[n]
    w = boxes[:, 2] - boxes[:, 0]
    h = boxes[:, 3] - boxes[:, 1]
    cx = boxes[:, 0] + 0.5 * w
    cy = boxes[:, 1] + 0.5 * h
    dx, dy = deltas[:, 0], deltas[:, 1]
    dw = jnp.minimum(deltas[:, 2], BBOX_CLIP)
    dh = jnp.minimum(deltas[:, 3], BBOX_CLIP)
    pcx = dx * w + cx
    pcy = dy * h + cy
    pw = jnp.exp(dw) * w
    ph = jnp.exp(dh) * h
    top_boxes = jnp.stack(
        [pcx - 0.5 * pw, pcy - 0.5 * ph, pcx + 0.5 * pw, pcy + 0.5 * ph], axis=-1)
    top_labels_f = (c + 1).astype(jnp.float32)
    return _nms(top_boxes, top_labels_f, top_scores, interpret)


def kernel(label_pre, bbox_pre, proposals):
    return _impl(label_pre, bbox_pre, proposals)


# speedup vs baseline: 6.7353x; 2.0230x over previous
import functools

import jax
import jax.numpy as jnp
import numpy as np
from jax import lax
from jax.experimental import pallas as pl
from jax.experimental.pallas import tpu as pltpu

N = 20000
C = 80
SCORE_THRESH = 0.01
NMS_THRESH = 0.5
DET_PER_IM = 100
PRE_NMS_TOPK = 1000
BBOX_CLIP = float(np.log(1000.0 / 16.0))

_ROWS = 1000
_PAD = 1024


def _score_kernel(lab_ref, dx_ref, dy_ref, dw_ref, dh_ref, prop_ref, out_ref):
    lab = lab_ref[...]
    m = jnp.max(lab, axis=1, keepdims=True)
    e = jnp.exp(lab - m)
    s = jnp.sum(e, axis=1, keepdims=True)
    scores = (e / s)[:, 1:]

    x1 = prop_ref[:, 0:1]
    y1 = prop_ref[:, 1:2]
    x2 = prop_ref[:, 2:3]
    y2 = prop_ref[:, 3:4]
    w = x2 - x1
    h = y2 - y1
    cx = x1 + 0.5 * w
    cy = y1 + 0.5 * h
    dx = dx_ref[...]
    dy = dy_ref[...]
    dwc = jnp.minimum(dw_ref[...], BBOX_CLIP)
    dhc = jnp.minimum(dh_ref[...], BBOX_CLIP)
    pcx = dx * w + cx
    pcy = dy * h + cy
    pw = jnp.exp(dwc) * w
    ph = jnp.exp(dhc) * h
    ax = (pcx + 0.5 * pw) - (pcx - 0.5 * pw)
    ay = (pcy + 0.5 * ph) - (pcy - 0.5 * ph)
    area = ay * ax
    valid = (scores > SCORE_THRESH) & (area > 0.1)
    out_ref[...] = jnp.where(valid, lax.bitcast_convert_type(scores, jnp.int32), 0)


def _nms_kernel(bx1_ref, by1_ref, bx2_ref, by2_ref, lab_ref, sc_ref, out_ref):
    jj = (lax.broadcasted_iota(jnp.int32, (8, 128), 0) * 128
          + lax.broadcasted_iota(jnp.int32, (8, 128), 1))
    real = jj < PRE_NMS_TOPK
    bx1 = bx1_ref[...]
    by1 = by1_ref[...]
    bx2 = bx2_ref[...]
    by2 = by2_ref[...]
    labs = lab_ref[...]
    scores = sc_ref[...]

    neg = jnp.float32(-jnp.inf)
    mc = jnp.maximum(
        jnp.max(jnp.where(real, bx1, neg)),
        jnp.maximum(
            jnp.max(jnp.where(real, by1, neg)),
            jnp.maximum(jnp.max(jnp.where(real, bx2, neg)),
                        jnp.max(jnp.where(real, by2, neg)))),
    ) + 1.0
    offs = labs * mc
    ox1 = bx1 + offs
    oy1 = by1 + offs
    ox2 = bx2 + offs
    oy2 = by2 + offs
    areas = (ox2 - ox1) * (oy2 - oy1)

    def ext(sel, a):
        return jnp.sum(jnp.where(sel, a, 0.0))

    def body(i, work):
        m = jnp.max(work)
        idx = jnp.min(jnp.where(work == m, jj, jnp.int32(1 << 30)))
        sel = jj == idx
        vx1 = ext(sel, ox1)
        vy1 = ext(sel, oy1)
        vx2 = ext(sel, ox2)
        vy2 = ext(sel, oy2)
        va = ext(sel, areas)
        w_i = jnp.sum(jnp.where(sel, work, 0.0))
        valid = w_i > 0.0
        xx1 = jnp.maximum(ox1, vx1)
        yy1 = jnp.maximum(oy1, vy1)
        xx2 = jnp.minimum(ox2, vx2)
        yy2 = jnp.minimum(oy2, vy2)
        inter = jnp.clip(xx2 - xx1, 0.0) * jnp.clip(yy2 - yy1, 0.0)
        iou = inter / (areas + va - inter + 1e-9)
        work = jnp.where(iou > NMS_THRESH, neg, work)
        li = lax.broadcasted_iota(jnp.int32, (1, 128), 1)
        row = jnp.zeros((1, 128), jnp.float32)
        for k, v in enumerate(
                (ext(sel, bx1), ext(sel, by1), ext(sel, bx2), ext(sel, by2),
                 jnp.sum(jnp.where(sel, scores, 0.0)))):
            row = jnp.where(li == k, jnp.where(valid, v, 0.0), row)
        out_ref[pl.ds(i, 1), :] = row
        return work

    lax.fori_loop(0, DET_PER_IM, body, scores)


def _masked_scores(label_pre, bbox_pre, proposals, interpret):
    dx = bbox_pre[:, 4::4]
    dy = bbox_pre[:, 5::4]
    dw = bbox_pre[:, 6::4]
    dh = bbox_pre[:, 7::4]
    grid = (N // _ROWS,)
    rb = lambda j: (j, 0)
    return pl.pallas_call(
        _score_kernel,
        grid=grid,
        in_specs=[
            pl.BlockSpec((_ROWS, C + 1), rb),
            pl.BlockSpec((_ROWS, C), rb),
            pl.BlockSpec((_ROWS, C), rb),
            pl.BlockSpec((_ROWS, C), rb),
            pl.BlockSpec((_ROWS, C), rb),
            pl.BlockSpec((_ROWS, 4), rb),
        ],
        out_specs=pl.BlockSpec((_ROWS, C), rb),
        out_shape=jax.ShapeDtypeStruct((N, C), jnp.int32),
        interpret=interpret,
    )(label_pre, dx, dy, dw, dh, proposals)


_KROWS = N * C // 128
_SLAB = 500
_NSLAB = _KROWS // _SLAB


def _select_kernel(keys_ref, idx_ref, key_ref):
    big = jnp.int32(1 << 30)

    def count_ge(x):
        def slab(i, acc):
            k = keys_ref[pl.ds(i * _SLAB, _SLAB), :]
            return acc + jnp.sum((k >= x).astype(jnp.int32))
        return lax.fori_loop(0, _NSLAB, slab, jnp.int32(0))

    def search(b, lo):
        x = lo + lax.shift_left(jnp.int32(1), 29 - b)
        return jnp.where(count_ge(x) >= PRE_NMS_TOPK, x, lo)

    lo = lax.fori_loop(0, 30, search, jnp.int32(0))
    m = count_ge(lo + 1)

    out_iota = (lax.broadcasted_iota(jnp.int32, (8, 128), 0) * 128
                + lax.broadcasted_iota(jnp.int32, (8, 128), 1))
    jj0 = out_iota

    def slab_body(s, carry):
        budget, total, acc_i, acc_k = carry
        k = keys_ref[pl.ds(s * 8, 8), :]
        jj = jj0 + s * 1024
        gt = k > lo
        eq = k == lo
        cand = (gt | (eq & (budget > 0))).astype(jnp.int32)

        def any_cand(c):
            return jnp.max(c[0]) > 0

        def extract(c):
            cand, budget, total, acc_i, acc_k = c
            candb = cand > 0
            i = jnp.min(jnp.where(candb, jj, big))
            sel = jj == i
            key_i = jnp.sum(jnp.where(sel, k, 0))
            budget = budget - jnp.where(key_i == lo, 1, 0)
            acc_i = jnp.where(out_iota == total, i, acc_i)
            acc_k = jnp.where(out_iota == total, key_i, acc_k)
            total = total + 1
            candb = (candb & (~sel)) & (gt | (eq & (budget > 0)))
            return candb.astype(jnp.int32), budget, total, acc_i, acc_k

        _, budget, total, acc_i, acc_k = lax.while_loop(
            any_cand, extract, (cand, budget, total, acc_i, acc_k))
        return budget, total, acc_i, acc_k

    zero = jnp.zeros((8, 128), jnp.int32)
    _, _, acc_i, acc_k = lax.fori_loop(
        0, _KROWS // 8, slab_body,
        (PRE_NMS_TOPK - m, jnp.int32(0), zero, zero))
    idx_ref[...] = acc_i
    key_ref[...] = acc_k


def _select(keys, interpret):
    return pl.pallas_call(
        _select_kernel,
        in_specs=[pl.BlockSpec((_KROWS, 128), lambda: (0, 0))],
        out_specs=[pl.BlockSpec((8, 128), lambda: (0, 0))] * 2,
        out_shape=[jax.ShapeDtypeStruct((8, 128), jnp.int32)] * 2,
        interpret=interpret,
    )(keys)


def _nms(top_boxes, top_labels_f, top_scores, interpret):
    npad = _PAD - PRE_NMS_TOPK

    def padv(a, v):
        return jnp.concatenate([a, jnp.full((npad,), v, a.dtype)]).reshape(8, 128)

    args = (
        padv(top_boxes[:, 0], 0.0),
        padv(top_boxes[:, 1], 0.0),
        padv(top_boxes[:, 2], 0.0),
        padv(top_boxes[:, 3], 0.0),
        padv(top_labels_f, 0.0),
        padv(top_scores, -jnp.inf),
    )
    out = pl.pallas_call(
        _nms_kernel,
        in_specs=[pl.BlockSpec((8, 128), lambda: (0, 0))] * 6,
        out_specs=pl.BlockSpec((DET_PER_IM, 128), lambda: (0, 0)),
        out_shape=jax.ShapeDtypeStruct((DET_PER_IM, 128), jnp.float32),
        interpret=interpret,
    )(*args)
    return out[:, :5]


def _impl(label_pre, bbox_pre, proposals, interpret=False):
    keys = _masked_scores(label_pre, bbox_pre, proposals, interpret)
    sel_idx, sel_key = _select(keys.reshape(_KROWS, 128), interpret)
    top_idx = sel_idx.reshape(-1)[:PRE_NMS_TOPK]
    key1k = sel_key.reshape(-1)[:PRE_NMS_TOPK]
    top_scores = jnp.where(
        key1k > 0, lax.bitcast_convert_type(key1k, jnp.float32), -1.0)
    n = top_idx // C
    c = top_idx % C
    deltas = bbox_pre[n[:, None], (4 * (c + 1))[:, None] + jnp.arange(4)[None, :]]
    boxes = proposals[n]
    w = boxes[:, 2] - boxes[:, 0]
    h = boxes[:, 3] - boxes[:, 1]
    cx = boxes[:, 0] + 0.5 * w
    cy = boxes[:, 1] + 0.5 * h
    dx, dy = deltas[:, 0], deltas[:, 1]
    dw = jnp.minimum(deltas[:, 2], BBOX_CLIP)
    dh = jnp.minimum(deltas[:, 3], BBOX_CLIP)
    pcx = dx * w + cx
    pcy = dy * h + cy
    pw = jnp.exp(dw) * w
    ph = jnp.exp(dh) * h
    top_boxes = jnp.stack(
        [pcx - 0.5 * pw, pcy - 0.5 * ph, pcx + 0.5 * pw, pcy + 0.5 * ph], axis=-1)
    top_labels_f = (c + 1).astype(jnp.float32)
    return _nms(top_boxes, top_labels_f, top_scores, interpret)


def kernel(label_pre, bbox_pre, proposals):
    return _impl(label_pre, bbox_pre, proposals)


# speedup vs baseline: 10.2821x; 1.5266x over previous
import functools

import jax
import jax.numpy as jnp
import numpy as np
from jax import lax
from jax.experimental import pallas as pl
from jax.experimental.pallas import tpu as pltpu

N = 20000
C = 80
SCORE_THRESH = 0.01
NMS_THRESH = 0.5
DET_PER_IM = 100
PRE_NMS_TOPK = 1000
BBOX_CLIP = float(np.log(1000.0 / 16.0))

_ROWS = 1000
_PAD = 1024


def _score_kernel(lab_ref, dx_ref, dy_ref, dw_ref, dh_ref, prop_ref, out_ref):
    lab = lab_ref[...]
    m = jnp.max(lab, axis=1, keepdims=True)
    e = jnp.exp(lab - m)
    s = jnp.sum(e, axis=1, keepdims=True)
    scores = (e / s)[:, 1:]

    x1 = prop_ref[:, 0:1]
    y1 = prop_ref[:, 1:2]
    x2 = prop_ref[:, 2:3]
    y2 = prop_ref[:, 3:4]
    w = x2 - x1
    h = y2 - y1
    cx = x1 + 0.5 * w
    cy = y1 + 0.5 * h
    dx = dx_ref[...]
    dy = dy_ref[...]
    dwc = jnp.minimum(dw_ref[...], BBOX_CLIP)
    dhc = jnp.minimum(dh_ref[...], BBOX_CLIP)
    pcx = dx * w + cx
    pcy = dy * h + cy
    pw = jnp.exp(dwc) * w
    ph = jnp.exp(dhc) * h
    ax = (pcx + 0.5 * pw) - (pcx - 0.5 * pw)
    ay = (pcy + 0.5 * ph) - (pcy - 0.5 * ph)
    area = ay * ax
    valid = (scores > SCORE_THRESH) & (area > 0.1)
    out_ref[...] = jnp.where(valid, lax.bitcast_convert_type(scores, jnp.int32), 0)


def _nms_kernel(bx1_ref, by1_ref, bx2_ref, by2_ref, lab_ref, sc_ref, out_ref):
    jj = (lax.broadcasted_iota(jnp.int32, (8, 128), 0) * 128
          + lax.broadcasted_iota(jnp.int32, (8, 128), 1))
    real = jj < PRE_NMS_TOPK
    bx1 = bx1_ref[...]
    by1 = by1_ref[...]
    bx2 = bx2_ref[...]
    by2 = by2_ref[...]
    labs = lab_ref[...]
    scores = sc_ref[...]

    neg = jnp.float32(-jnp.inf)
    mc = jnp.maximum(
        jnp.max(jnp.where(real, bx1, neg)),
        jnp.maximum(
            jnp.max(jnp.where(real, by1, neg)),
            jnp.maximum(jnp.max(jnp.where(real, bx2, neg)),
                        jnp.max(jnp.where(real, by2, neg)))),
    ) + 1.0
    offs = labs * mc
    ox1 = bx1 + offs
    oy1 = by1 + offs
    ox2 = bx2 + offs
    oy2 = by2 + offs
    areas = (ox2 - ox1) * (oy2 - oy1)

    def ext(sel, a):
        return jnp.sum(jnp.where(sel, a, 0.0))

    def body(i, work):
        m = jnp.max(work)
        idx = jnp.min(jnp.where(work == m, jj, jnp.int32(1 << 30)))
        sel = jj == idx
        vx1 = ext(sel, ox1)
        vy1 = ext(sel, oy1)
        vx2 = ext(sel, ox2)
        vy2 = ext(sel, oy2)
        va = ext(sel, areas)
        w_i = jnp.sum(jnp.where(sel, work, 0.0))
        valid = w_i > 0.0
        xx1 = jnp.maximum(ox1, vx1)
        yy1 = jnp.maximum(oy1, vy1)
        xx2 = jnp.minimum(ox2, vx2)
        yy2 = jnp.minimum(oy2, vy2)
        inter = jnp.clip(xx2 - xx1, 0.0) * jnp.clip(yy2 - yy1, 0.0)
        iou = inter / (areas + va - inter + 1e-9)
        work = jnp.where(iou > NMS_THRESH, neg, work)
        li = lax.broadcasted_iota(jnp.int32, (1, 128), 1)
        row = jnp.zeros((1, 128), jnp.float32)
        for k, v in enumerate(
                (ext(sel, bx1), ext(sel, by1), ext(sel, bx2), ext(sel, by2),
                 jnp.sum(jnp.where(sel, scores, 0.0)))):
            row = jnp.where(li == k, jnp.where(valid, v, 0.0), row)
        out_ref[pl.ds(i, 1), :] = row
        return work

    lax.fori_loop(0, DET_PER_IM, body, scores)


def _masked_scores(label_pre, bbox_pre, proposals, interpret):
    dx = bbox_pre[:, 4::4]
    dy = bbox_pre[:, 5::4]
    dw = bbox_pre[:, 6::4]
    dh = bbox_pre[:, 7::4]
    grid = (N // _ROWS,)
    rb = lambda j: (j, 0)
    return pl.pallas_call(
        _score_kernel,
        grid=grid,
        in_specs=[
            pl.BlockSpec((_ROWS, C + 1), rb),
            pl.BlockSpec((_ROWS, C), rb),
            pl.BlockSpec((_ROWS, C), rb),
            pl.BlockSpec((_ROWS, C), rb),
            pl.BlockSpec((_ROWS, C), rb),
            pl.BlockSpec((_ROWS, 4), rb),
        ],
        out_specs=pl.BlockSpec((_ROWS, C), rb),
        out_shape=jax.ShapeDtypeStruct((N, C), jnp.int32),
        interpret=interpret,
    )(label_pre, dx, dy, dw, dh, proposals)


_KROWS = N * C // 128
_SLAB = 500
_NSLAB = _KROWS // _SLAB
_XS = 100
_NX = _KROWS // _XS


def _select_kernel(keys_ref, idx_ref, key_ref, g_ref, e_ref, c_ref):
    big = jnp.int32(1 << 30)

    def count_ge(x):
        def slab(i, acc):
            k = keys_ref[pl.ds(i * _SLAB, _SLAB), :]
            return acc + jnp.sum((k >= x).astype(jnp.int32))
        return lax.fori_loop(0, _NSLAB, slab, jnp.int32(0))

    def search(b, lo):
        x = lo + lax.shift_left(jnp.int32(1), 29 - b)
        return jnp.where(count_ge(x) >= PRE_NMS_TOPK, x, lo)

    lo = lax.fori_loop(0, 30, search, jnp.int32(0))

    def stats(s, macc):
        k = keys_ref[pl.ds(s * _XS, _XS), :]
        g = jnp.sum((k > lo).astype(jnp.int32))
        g_ref[s] = g
        e_ref[s] = jnp.sum((k == lo).astype(jnp.int32))
        return macc + g

    m = lax.fori_loop(0, _NX, stats, jnp.int32(0))
    r = PRE_NMS_TOPK - m

    def pref(s, eqp):
        e_s = e_ref[s]
        take = jnp.clip(r - eqp, 0, e_s)
        c_ref[s] = g_ref[s] + take
        return eqp + e_s

    lax.fori_loop(0, _NX, pref, jnp.int32(0))

    out_iota = (lax.broadcasted_iota(jnp.int32, (8, 128), 0) * 128
                + lax.broadcasted_iota(jnp.int32, (8, 128), 1))
    jx = (lax.broadcasted_iota(jnp.int32, (_XS, 128), 0) * 128
          + lax.broadcasted_iota(jnp.int32, (_XS, 128), 1))

    def slab_body(s, carry):
        budget, total, acc_i, acc_k = carry
        k = keys_ref[pl.ds(s * _XS, _XS), :]
        jj = jx + s * (_XS * 128)
        gt = k > lo
        eq = k == lo
        cand = (gt | (eq & (budget > 0))).astype(jnp.int32)

        def extract(_, c):
            cand, budget, total, acc_i, acc_k = c
            candb = cand > 0
            i = jnp.min(jnp.where(candb, jj, big))
            sel = jj == i
            key_i = jnp.sum(jnp.where(sel, k, 0))
            budget = budget - jnp.where(key_i == lo, 1, 0)
            acc_i = jnp.where(out_iota == total, i, acc_i)
            acc_k = jnp.where(out_iota == total, key_i, acc_k)
            total = total + 1
            candb = (candb & (~sel)) & (gt | (eq & (budget > 0)))
            return candb.astype(jnp.int32), budget, total, acc_i, acc_k

        _, budget, total, acc_i, acc_k = lax.fori_loop(
            0, c_ref[s], extract, (cand, budget, total, acc_i, acc_k))
        return budget, total, acc_i, acc_k

    zero = jnp.zeros((8, 128), jnp.int32)
    _, _, acc_i, acc_k = lax.fori_loop(
        0, _NX, slab_body, (r, jnp.int32(0), zero, zero))
    idx_ref[...] = acc_i
    key_ref[...] = acc_k


def _select(keys, interpret):
    return pl.pallas_call(
        _select_kernel,
        in_specs=[pl.BlockSpec((_KROWS, 128), lambda: (0, 0))],
        out_specs=[pl.BlockSpec((8, 128), lambda: (0, 0))] * 2,
        out_shape=[jax.ShapeDtypeStruct((8, 128), jnp.int32)] * 2,
        scratch_shapes=[pltpu.SMEM((_NX,), jnp.int32)] * 3,
        interpret=interpret,
    )(keys)


def _nms(top_boxes, top_labels_f, top_scores, interpret):
    npad = _PAD - PRE_NMS_TOPK

    def padv(a, v):
        return jnp.concatenate([a, jnp.full((npad,), v, a.dtype)]).reshape(8, 128)

    args = (
        padv(top_boxes[:, 0], 0.0),
        padv(top_boxes[:, 1], 0.0),
        padv(top_boxes[:, 2], 0.0),
        padv(top_boxes[:, 3], 0.0),
        padv(top_labels_f, 0.0),
        padv(top_scores, -jnp.inf),
    )
    out = pl.pallas_call(
        _nms_kernel,
        in_specs=[pl.BlockSpec((8, 128), lambda: (0, 0))] * 6,
        out_specs=pl.BlockSpec((DET_PER_IM, 128), lambda: (0, 0)),
        out_shape=jax.ShapeDtypeStruct((DET_PER_IM, 128), jnp.float32),
        interpret=interpret,
    )(*args)
    return out[:, :5]


def _impl(label_pre, bbox_pre, proposals, interpret=False):
    keys = _masked_scores(label_pre, bbox_pre, proposals, interpret)
    sel_idx, sel_key = _select(keys.reshape(_KROWS, 128), interpret)
    top_idx = sel_idx.reshape(-1)[:PRE_NMS_TOPK]
    key1k = sel_key.reshape(-1)[:PRE_NMS_TOPK]
    top_scores = jnp.where(
        key1k > 0, lax.bitcast_convert_type(key1k, jnp.float32), -1.0)
    n = top_idx // C
    c = top_idx % C
    deltas = bbox_pre[n[:, None], (4 * (c + 1))[:, None] + jnp.arange(4)[None, :]]
    boxes = proposals[n]
    w = boxes[:, 2] - boxes[:, 0]
    h = boxes[:, 3] - boxes[:, 1]
    cx = boxes[:, 0] + 0.5 * w
    cy = boxes[:, 1] + 0.5 * h
    dx, dy = deltas[:, 0], deltas[:, 1]
    dw = jnp.minimum(deltas[:, 2], BBOX_CLIP)
    dh = jnp.minimum(deltas[:, 3], BBOX_CLIP)
    pcx = dx * w + cx
    pcy = dy * h + cy
    pw = jnp.exp(dw) * w
    ph = jnp.exp(dh) * h
    top_boxes = jnp.stack(
        [pcx - 0.5 * pw, pcy - 0.5 * ph, pcx + 0.5 * pw, pcy + 0.5 * ph], axis=-1)
    top_labels_f = (c + 1).astype(jnp.float32)
    return _nms(top_boxes, top_labels_f, top_scores, interpret)


def kernel(label_pre, bbox_pre, proposals):
    return _impl(label_pre, bbox_pre, proposals)


# speedup vs baseline: 11.9295x; 1.1602x over previous
import functools

import jax
import jax.numpy as jnp
import numpy as np
from jax import lax
from jax.experimental import pallas as pl
from jax.experimental.pallas import tpu as pltpu

N = 20000
C = 80
SCORE_THRESH = 0.01
NMS_THRESH = 0.5
DET_PER_IM = 100
PRE_NMS_TOPK = 1000
BBOX_CLIP = float(np.log(1000.0 / 16.0))

_ROWS = 1000
_PAD = 1024


def _score_kernel(lab_ref, dx_ref, dy_ref, dw_ref, dh_ref, prop_ref, out_ref):
    lab = lab_ref[...]
    m = jnp.max(lab, axis=1, keepdims=True)
    e = jnp.exp(lab - m)
    s = jnp.sum(e, axis=1, keepdims=True)
    scores = (e / s)[:, 1:]

    x1 = prop_ref[:, 0:1]
    y1 = prop_ref[:, 1:2]
    x2 = prop_ref[:, 2:3]
    y2 = prop_ref[:, 3:4]
    w = x2 - x1
    h = y2 - y1
    cx = x1 + 0.5 * w
    cy = y1 + 0.5 * h
    dx = dx_ref[...]
    dy = dy_ref[...]
    dwc = jnp.minimum(dw_ref[...], BBOX_CLIP)
    dhc = jnp.minimum(dh_ref[...], BBOX_CLIP)
    pcx = dx * w + cx
    pcy = dy * h + cy
    pw = jnp.exp(dwc) * w
    ph = jnp.exp(dhc) * h
    ax = (pcx + 0.5 * pw) - (pcx - 0.5 * pw)
    ay = (pcy + 0.5 * ph) - (pcy - 0.5 * ph)
    area = ay * ax
    valid = (scores > SCORE_THRESH) & (area > 0.1)
    out_ref[...] = jnp.where(valid, lax.bitcast_convert_type(scores, jnp.int32), 0)


def _nms_kernel(bx1_ref, by1_ref, bx2_ref, by2_ref, lab_ref, sc_ref, out_ref):
    jj = (lax.broadcasted_iota(jnp.int32, (8, 128), 0) * 128
          + lax.broadcasted_iota(jnp.int32, (8, 128), 1))
    real = jj < PRE_NMS_TOPK
    bx1 = bx1_ref[...]
    by1 = by1_ref[...]
    bx2 = bx2_ref[...]
    by2 = by2_ref[...]
    labs = lab_ref[...]
    scores = sc_ref[...]

    neg = jnp.float32(-jnp.inf)
    mc = jnp.maximum(
        jnp.max(jnp.where(real, bx1, neg)),
        jnp.maximum(
            jnp.max(jnp.where(real, by1, neg)),
            jnp.maximum(jnp.max(jnp.where(real, bx2, neg)),
                        jnp.max(jnp.where(real, by2, neg)))),
    ) + 1.0
    offs = labs * mc
    ox1 = bx1 + offs
    oy1 = by1 + offs
    ox2 = bx2 + offs
    oy2 = by2 + offs
    areas = (ox2 - ox1) * (oy2 - oy1)

    def ext(sel, a):
        return jnp.sum(jnp.where(sel, a, 0.0))

    def body(i, work):
        m = jnp.max(work)
        idx = jnp.min(jnp.where(work == m, jj, jnp.int32(1 << 30)))
        sel = jj == idx
        vx1 = ext(sel, ox1)
        vy1 = ext(sel, oy1)
        vx2 = ext(sel, ox2)
        vy2 = ext(sel, oy2)
        va = ext(sel, areas)
        w_i = jnp.sum(jnp.where(sel, work, 0.0))
        valid = w_i > 0.0
        xx1 = jnp.maximum(ox1, vx1)
        yy1 = jnp.maximum(oy1, vy1)
        xx2 = jnp.minimum(ox2, vx2)
        yy2 = jnp.minimum(oy2, vy2)
        inter = jnp.clip(xx2 - xx1, 0.0) * jnp.clip(yy2 - yy1, 0.0)
        iou = inter / (areas + va - inter + 1e-9)
        work = jnp.where(iou > NMS_THRESH, neg, work)
        li = lax.broadcasted_iota(jnp.int32, (1, 128), 1)
        row = jnp.zeros((1, 128), jnp.float32)
        for k, v in enumerate(
                (ext(sel, bx1), ext(sel, by1), ext(sel, bx2), ext(sel, by2),
                 jnp.sum(jnp.where(sel, scores, 0.0)))):
            row = jnp.where(li == k, jnp.where(valid, v, 0.0), row)
        out_ref[pl.ds(i, 1), :] = row
        return work

    lax.fori_loop(0, DET_PER_IM, body, scores)


def _masked_scores(label_pre, bbox_pre, proposals, interpret):
    dx = bbox_pre[:, 4::4]
    dy = bbox_pre[:, 5::4]
    dw = bbox_pre[:, 6::4]
    dh = bbox_pre[:, 7::4]
    grid = (N // _ROWS,)
    rb = lambda j: (j, 0)
    return pl.pallas_call(
        _score_kernel,
        grid=grid,
        in_specs=[
            pl.BlockSpec((_ROWS, C + 1), rb),
            pl.BlockSpec((_ROWS, C), rb),
            pl.BlockSpec((_ROWS, C), rb),
            pl.BlockSpec((_ROWS, C), rb),
            pl.BlockSpec((_ROWS, C), rb),
            pl.BlockSpec((_ROWS, 4), rb),
        ],
        out_specs=pl.BlockSpec((_ROWS, C), rb),
        out_shape=jax.ShapeDtypeStruct((N, C), jnp.int32),
        interpret=interpret,
    )(label_pre, dx, dy, dw, dh, proposals)


_KROWS = N * C // 128
_SLAB = 500
_NSLAB = _KROWS // _SLAB
_XS = 50
_NX = _KROWS // _XS
_KEY_FLOOR = 0x3C000000


def _select_kernel(keys_ref, idx_ref, g_ref, e_ref, c_ref):
    big = jnp.int32(1 << 30)

    def count_ge(x):
        def slab(i, acc):
            k = keys_ref[pl.ds(i * _SLAB, _SLAB), :]
            return acc + jnp.sum((k >= x).astype(jnp.int32))
        return lax.fori_loop(0, _NSLAB, slab, jnp.int32(0))

    def search(b, lo):
        x = lo + lax.shift_left(jnp.int32(1), 25 - b)
        return jnp.where(count_ge(x) >= PRE_NMS_TOPK, x, lo)

    lo0 = jnp.where(count_ge(jnp.int32(_KEY_FLOOR)) >= PRE_NMS_TOPK,
                    jnp.int32(_KEY_FLOOR), jnp.int32(0))
    lo = lax.fori_loop(0, 26, search, lo0)

    def stats(s, macc):
        k = keys_ref[pl.ds(s * _XS, _XS), :]
        g = jnp.sum((k > lo).astype(jnp.int32))
        g_ref[s] = g
        e_ref[s] = jnp.sum((k == lo).astype(jnp.int32))
        return macc + g

    m = lax.fori_loop(0, _NX, stats, jnp.int32(0))
    r = PRE_NMS_TOPK - m

    def pref(s, eqp):
        e_s = e_ref[s]
        take = jnp.clip(r - eqp, 0, e_s)
        c_ref[s] = g_ref[s] + take
        return eqp + e_s

    lax.fori_loop(0, _NX, pref, jnp.int32(0))

    out_iota = (lax.broadcasted_iota(jnp.int32, (8, 128), 0) * 128
                + lax.broadcasted_iota(jnp.int32, (8, 128), 1))
    jx = (lax.broadcasted_iota(jnp.int32, (_XS, 128), 0) * 128
          + lax.broadcasted_iota(jnp.int32, (_XS, 128), 1))

    def slab_body(s, carry):
        budget, total, acc_i = carry
        k = keys_ref[pl.ds(s * _XS, _XS), :]
        jj = jx + s * (_XS * 128)
        gt = k > lo
        eq = k == lo
        cc = jj * 2 + eq.astype(jnp.int32)
        cand = (gt | (eq & (budget > 0))).astype(jnp.int32)

        def extract(_, c):
            cand, budget, total, acc_i = c
            candb = cand > 0
            i = jnp.min(jnp.where(candb, cc, big))
            budget = budget - (i & 1)
            acc_i = jnp.where(out_iota == total, lax.shift_right_logical(i, 1),
                              acc_i)
            total = total + 1
            candb = (candb & (cc != i)) & (gt | (eq & (budget > 0)))
            return candb.astype(jnp.int32), budget, total, acc_i

        _, budget, total, acc_i = lax.fori_loop(
            0, c_ref[s], extract, (cand, budget, total, acc_i))
        return budget, total, acc_i

    zero = jnp.zeros((8, 128), jnp.int32)
    _, _, acc_i = lax.fori_loop(
        0, _NX, slab_body, (r, jnp.int32(0), zero))
    idx_ref[...] = acc_i


def _select(keys, interpret):
    return pl.pallas_call(
        _select_kernel,
        in_specs=[pl.BlockSpec((_KROWS, 128), lambda: (0, 0))],
        out_specs=pl.BlockSpec((8, 128), lambda: (0, 0)),
        out_shape=jax.ShapeDtypeStruct((8, 128), jnp.int32),
        scratch_shapes=[pltpu.SMEM((_NX,), jnp.int32)] * 3,
        interpret=interpret,
    )(keys)


def _nms(top_boxes, top_labels_f, top_scores, interpret):
    npad = _PAD - PRE_NMS_TOPK

    def padv(a, v):
        return jnp.concatenate([a, jnp.full((npad,), v, a.dtype)]).reshape(8, 128)

    args = (
        padv(top_boxes[:, 0], 0.0),
        padv(top_boxes[:, 1], 0.0),
        padv(top_boxes[:, 2], 0.0),
        padv(top_boxes[:, 3], 0.0),
        padv(top_labels_f, 0.0),
        padv(top_scores, -jnp.inf),
    )
    out = pl.pallas_call(
        _nms_kernel,
        in_specs=[pl.BlockSpec((8, 128), lambda: (0, 0))] * 6,
        out_specs=pl.BlockSpec((DET_PER_IM, 128), lambda: (0, 0)),
        out_shape=jax.ShapeDtypeStruct((DET_PER_IM, 128), jnp.float32),
        interpret=interpret,
    )(*args)
    return out[:, :5]


def _impl(label_pre, bbox_pre, proposals, interpret=False):
    keys = _masked_scores(label_pre, bbox_pre, proposals, interpret)
    sel_idx = _select(keys.reshape(_KROWS, 128), interpret)
    top_idx = sel_idx.reshape(-1)[:PRE_NMS_TOPK]
    key1k = keys.reshape(-1)[top_idx]
    top_scores = jnp.where(
        key1k > 0, lax.bitcast_convert_type(key1k, jnp.float32), -1.0)
    n = top_idx // C
    c = top_idx % C
    deltas = bbox_pre[n[:, None], (4 * (c + 1))[:, None] + jnp.arange(4)[None, :]]
    boxes = proposals[n]
    w = boxes[:, 2] - boxes[:, 0]
    h = boxes[:, 3] - boxes[:, 1]
    cx = boxes[:, 0] + 0.5 * w
    cy = boxes[:, 1] + 0.5 * h
    dx, dy = deltas[:, 0], deltas[:, 1]
    dw = jnp.minimum(deltas[:, 2], BBOX_CLIP)
    dh = jnp.minimum(deltas[:, 3], BBOX_CLIP)
    pcx = dx * w + cx
    pcy = dy * h + cy
    pw = jnp.exp(dw) * w
    ph = jnp.exp(dh) * h
    top_boxes = jnp.stack(
        [pcx - 0.5 * pw, pcy - 0.5 * ph, pcx + 0.5 * pw, pcy + 0.5 * ph], axis=-1)
    top_labels_f = (c + 1).astype(jnp.float32)
    return _nms(top_boxes, top_labels_f, top_scores, interpret)


def kernel(label_pre, bbox_pre, proposals):
    return _impl(label_pre, bbox_pre, proposals)


# speedup vs baseline: 12.5255x; 1.0500x over previous
import functools

import jax
import jax.numpy as jnp
import numpy as np
from jax import lax
from jax.experimental import pallas as pl
from jax.experimental.pallas import tpu as pltpu

N = 20000
C = 80
SCORE_THRESH = 0.01
NMS_THRESH = 0.5
DET_PER_IM = 100
PRE_NMS_TOPK = 1000
BBOX_CLIP = float(np.log(1000.0 / 16.0))

_ROWS = 1000
_PAD = 1024


def _score_kernel(lab_ref, dx_ref, dy_ref, dw_ref, dh_ref, prop_ref, out_ref):
    lab = lab_ref[...]
    m = jnp.max(lab, axis=1, keepdims=True)
    e = jnp.exp(lab - m)
    s = jnp.sum(e, axis=1, keepdims=True)
    scores = (e / s)[:, 1:]

    x1 = prop_ref[:, 0:1]
    y1 = prop_ref[:, 1:2]
    x2 = prop_ref[:, 2:3]
    y2 = prop_ref[:, 3:4]
    w = x2 - x1
    h = y2 - y1
    cx = x1 + 0.5 * w
    cy = y1 + 0.5 * h
    dx = dx_ref[...]
    dy = dy_ref[...]
    dwc = jnp.minimum(dw_ref[...], BBOX_CLIP)
    dhc = jnp.minimum(dh_ref[...], BBOX_CLIP)
    pcx = dx * w + cx
    pcy = dy * h + cy
    pw = jnp.exp(dwc) * w
    ph = jnp.exp(dhc) * h
    ax = (pcx + 0.5 * pw) - (pcx - 0.5 * pw)
    ay = (pcy + 0.5 * ph) - (pcy - 0.5 * ph)
    area = ay * ax
    valid = (scores > SCORE_THRESH) & (area > 0.1)
    out_ref[...] = jnp.where(valid, lax.bitcast_convert_type(scores, jnp.int32), 0)


def _nms_kernel(bx1_ref, by1_ref, bx2_ref, by2_ref, lab_ref, sc_ref, out_ref):
    jj = (lax.broadcasted_iota(jnp.int32, (8, 128), 0) * 128
          + lax.broadcasted_iota(jnp.int32, (8, 128), 1))
    real = jj < PRE_NMS_TOPK
    bx1 = bx1_ref[...]
    by1 = by1_ref[...]
    bx2 = bx2_ref[...]
    by2 = by2_ref[...]
    labs = lab_ref[...]
    scores = sc_ref[...]

    neg = jnp.float32(-jnp.inf)
    mc = jnp.maximum(
        jnp.max(jnp.where(real, bx1, neg)),
        jnp.maximum(
            jnp.max(jnp.where(real, by1, neg)),
            jnp.maximum(jnp.max(jnp.where(real, bx2, neg)),
                        jnp.max(jnp.where(real, by2, neg)))),
    ) + 1.0
    offs = labs * mc
    ox1 = bx1 + offs
    oy1 = by1 + offs
    ox2 = bx2 + offs
    oy2 = by2 + offs
    areas = (ox2 - ox1) * (oy2 - oy1)

    def ext(sel, a):
        return jnp.sum(jnp.where(sel, a, 0.0))

    def body(i, work):
        m = jnp.max(work)
        idx = jnp.min(jnp.where(work == m, jj, jnp.int32(1 << 30)))
        sel = jj == idx
        vx1 = ext(sel, ox1)
        vy1 = ext(sel, oy1)
        vx2 = ext(sel, ox2)
        vy2 = ext(sel, oy2)
        va = ext(sel, areas)
        w_i = jnp.sum(jnp.where(sel, work, 0.0))
        valid = w_i > 0.0
        xx1 = jnp.maximum(ox1, vx1)
        yy1 = jnp.maximum(oy1, vy1)
        xx2 = jnp.minimum(ox2, vx2)
        yy2 = jnp.minimum(oy2, vy2)
        inter = jnp.clip(xx2 - xx1, 0.0) * jnp.clip(yy2 - yy1, 0.0)
        iou = inter / (areas + va - inter + 1e-9)
        work = jnp.where(iou > NMS_THRESH, neg, work)
        li = lax.broadcasted_iota(jnp.int32, (1, 128), 1)
        row = jnp.zeros((1, 128), jnp.float32)
        for k, v in enumerate(
                (ext(sel, bx1), ext(sel, by1), ext(sel, bx2), ext(sel, by2),
                 jnp.sum(jnp.where(sel, scores, 0.0)))):
            row = jnp.where(li == k, jnp.where(valid, v, 0.0), row)
        out_ref[pl.ds(i, 1), :] = row
        return work

    lax.fori_loop(0, DET_PER_IM, body, scores)


def _masked_scores(label_pre, bbox_pre, proposals, interpret):
    dx = bbox_pre[:, 4::4]
    dy = bbox_pre[:, 5::4]
    dw = bbox_pre[:, 6::4]
    dh = bbox_pre[:, 7::4]
    grid = (N // _ROWS,)
    rb = lambda j: (j, 0)
    return pl.pallas_call(
        _score_kernel,
        grid=grid,
        in_specs=[
            pl.BlockSpec((_ROWS, C + 1), rb),
            pl.BlockSpec((_ROWS, C), rb),
            pl.BlockSpec((_ROWS, C), rb),
            pl.BlockSpec((_ROWS, C), rb),
            pl.BlockSpec((_ROWS, C), rb),
            pl.BlockSpec((_ROWS, 4), rb),
        ],
        out_specs=pl.BlockSpec((_ROWS, C), rb),
        out_shape=jax.ShapeDtypeStruct((N, C), jnp.int32),
        interpret=interpret,
    )(label_pre, dx, dy, dw, dh, proposals)


_KROWS = N * C // 128
_SLAB = 500
_NSLAB = _KROWS // _SLAB
_XS = 50
_NX = _KROWS // _XS
_KEY_FLOOR = 0x3C000000


def _select_kernel(keys_ref, idx_ref, g_ref, e_ref, c_ref):
    big = jnp.int32(1 << 30)

    def count_ge(x):
        def slab(i, acc):
            k = keys_ref[pl.ds(i * _SLAB, _SLAB), :]
            return acc + jnp.sum((k >= x).astype(jnp.int32))
        return lax.fori_loop(0, _NSLAB, slab, jnp.int32(0))

    def search4(b, lo):
        q = lax.shift_left(jnp.int32(1), 24 - 2 * b)

        def slab(i, acc):
            k = keys_ref[pl.ds(i * _SLAB, _SLAB), :]
            c1, c2, c3 = acc
            return (c1 + jnp.sum((k >= lo + q).astype(jnp.int32)),
                    c2 + jnp.sum((k >= lo + 2 * q).astype(jnp.int32)),
                    c3 + jnp.sum((k >= lo + 3 * q).astype(jnp.int32)))

        z = jnp.int32(0)
        c1, c2, c3 = lax.fori_loop(0, _NSLAB, slab, (z, z, z))
        kk = jnp.int32(PRE_NMS_TOPK)
        step = jnp.where(c3 >= kk, 3 * q,
                         jnp.where(c2 >= kk, 2 * q,
                                   jnp.where(c1 >= kk, q, 0)))
        return lo + step

    lo0 = jnp.where(count_ge(jnp.int32(_KEY_FLOOR)) >= PRE_NMS_TOPK,
                    jnp.int32(_KEY_FLOOR), jnp.int32(0))
    lo = lax.fori_loop(0, 13, search4, lo0)

    def stats(s, macc):
        k = keys_ref[pl.ds(s * _XS, _XS), :]
        g = jnp.sum((k > lo).astype(jnp.int32))
        g_ref[s] = g
        e_ref[s] = jnp.sum((k == lo).astype(jnp.int32))
        return macc + g

    m = lax.fori_loop(0, _NX, stats, jnp.int32(0))
    r = PRE_NMS_TOPK - m

    def pref(s, eqp):
        e_s = e_ref[s]
        take = jnp.clip(r - eqp, 0, e_s)
        c_ref[s] = g_ref[s] + take
        return eqp + e_s

    lax.fori_loop(0, _NX, pref, jnp.int32(0))

    out_iota = (lax.broadcasted_iota(jnp.int32, (8, 128), 0) * 128
                + lax.broadcasted_iota(jnp.int32, (8, 128), 1))
    jx = (lax.broadcasted_iota(jnp.int32, (_XS, 128), 0) * 128
          + lax.broadcasted_iota(jnp.int32, (_XS, 128), 1))

    def slab_body(s, carry):
        budget, total, acc_i = carry
        k = keys_ref[pl.ds(s * _XS, _XS), :]
        jj = jx + s * (_XS * 128)
        gt = k > lo
        eq = k == lo
        cc = jj * 2 + eq.astype(jnp.int32)
        cand = (gt | (eq & (budget > 0))).astype(jnp.int32)

        def extract(_, c):
            cand, budget, total, acc_i = c
            candb = cand > 0
            i = jnp.min(jnp.where(candb, cc, big), axis=(0, 1), keepdims=True)
            budget = budget - (i & 1)
            acc_i = jnp.where(out_iota == total, lax.shift_right_logical(i, 1),
                              acc_i)
            total = total + 1
            candb = (candb & (cc != i)) & (gt | (eq & (budget > 0)))
            return candb.astype(jnp.int32), budget, total, acc_i

        _, budget, total, acc_i = lax.fori_loop(
            0, c_ref[s], extract, (cand, budget, total, acc_i))
        return budget, total, acc_i

    zero = jnp.zeros((8, 128), jnp.int32)
    _, _, acc_i = lax.fori_loop(
        0, _NX, slab_body,
        (jnp.full((1, 1), r, jnp.int32), jnp.zeros((1, 1), jnp.int32), zero))
    idx_ref[...] = acc_i


def _select(keys, interpret):
    return pl.pallas_call(
        _select_kernel,
        in_specs=[pl.BlockSpec((_KROWS, 128), lambda: (0, 0))],
        out_specs=pl.BlockSpec((8, 128), lambda: (0, 0)),
        out_shape=jax.ShapeDtypeStruct((8, 128), jnp.int32),
        scratch_shapes=[pltpu.SMEM((_NX,), jnp.int32)] * 3,
        interpret=interpret,
    )(keys)


def _nms(top_boxes, top_labels_f, top_scores, interpret):
    npad = _PAD - PRE_NMS_TOPK

    def padv(a, v):
        return jnp.concatenate([a, jnp.full((npad,), v, a.dtype)]).reshape(8, 128)

    args = (
        padv(top_boxes[:, 0], 0.0),
        padv(top_boxes[:, 1], 0.0),
        padv(top_boxes[:, 2], 0.0),
        padv(top_boxes[:, 3], 0.0),
        padv(top_labels_f, 0.0),
        padv(top_scores, -jnp.inf),
    )
    out = pl.pallas_call(
        _nms_kernel,
        in_specs=[pl.BlockSpec((8, 128), lambda: (0, 0))] * 6,
        out_specs=pl.BlockSpec((DET_PER_IM, 128), lambda: (0, 0)),
        out_shape=jax.ShapeDtypeStruct((DET_PER_IM, 128), jnp.float32),
        interpret=interpret,
    )(*args)
    return out[:, :5]


def _impl(label_pre, bbox_pre, proposals, interpret=False):
    keys = _masked_scores(label_pre, bbox_pre, proposals, interpret)
    sel_idx = _select(keys.reshape(_KROWS, 128), interpret)
    top_idx = sel_idx.reshape(-1)[:PRE_NMS_TOPK]
    key1k = keys.reshape(-1)[top_idx]
    top_scores = jnp.where(
        key1k > 0, lax.bitcast_convert_type(key1k, jnp.float32), -1.0)
    n = top_idx // C
    c = top_idx % C
    deltas = bbox_pre[n[:, None], (4 * (c + 1))[:, None] + jnp.arange(4)[None, :]]
    boxes = proposals[n]
    w = boxes[:, 2] - boxes[:, 0]
    h = boxes[:, 3] - boxes[:, 1]
    cx = boxes[:, 0] + 0.5 * w
    cy = boxes[:, 1] + 0.5 * h
    dx, dy = deltas[:, 0], deltas[:, 1]
    dw = jnp.minimum(deltas[:, 2], BBOX_CLIP)
    dh = jnp.minimum(deltas[:, 3], BBOX_CLIP)
    pcx = dx * w + cx
    pcy = dy * h + cy
    pw = jnp.exp(dw) * w
    ph = jnp.exp(dh) * h
    top_boxes = jnp.stack(
        [pcx - 0.5 * pw, pcy - 0.5 * ph, pcx + 0.5 * pw, pcy + 0.5 * ph], axis=-1)
    top_labels_f = (c + 1).astype(jnp.float32)
    return _nms(top_boxes, top_labels_f, top_scores, interpret)


def kernel(label_pre, bbox_pre, proposals):
    return _impl(label_pre, bbox_pre, proposals)
